# baseline (speedup 1.0000x reference)
import jax
import jax.numpy as jnp
from jax.experimental import pallas as pl
from jax.experimental.pallas import tpu as pltpu

N = 100000
D_IN = 128
D_OUT = 128
BM = 5000


def _gconv_body(x_ref, w_ref, a_ref, o_ref, c_ref):
    @pl.when(pl.program_id(0) == 0)
    def _():
        c_ref[...] = jnp.dot(w_ref[...], a_ref[...],
                             preferred_element_type=jnp.float32)

    o_ref[...] = jnp.dot(x_ref[...], c_ref[...],
                         preferred_element_type=jnp.float32)


@jax.jit
def kernel(x, W, adj):
    grid = (N // BM,)
    return pl.pallas_call(
        _gconv_body,
        grid=grid,
        in_specs=[
            pl.BlockSpec((BM, D_IN), lambda i: (i, 0)),
            pl.BlockSpec((D_IN, D_OUT), lambda i: (0, 0)),
            pl.BlockSpec((D_OUT, D_OUT), lambda i: (0, 0)),
        ],
        out_specs=pl.BlockSpec((BM, D_OUT), lambda i: (i, 0)),
        out_shape=jax.ShapeDtypeStruct((N, D_OUT), jnp.float32),
        scratch_shapes=[pltpu.VMEM((D_IN, D_OUT), jnp.float32)],
        compiler_params=pltpu.CompilerParams(
            dimension_semantics=("arbitrary",),
        ),
    )(x, W, adj)
